# baseline (speedup 1.0000x reference)
import functools

import jax
import jax.numpy as jnp
from jax import lax
from jax.experimental import pallas as pl
from jax.experimental.pallas import tpu as pltpu
from jax.experimental.pallas import tpu_sc as plsc

_B, _N, _D, _H = 2, 2048, 256, 4
_HD = _D // _H
_KTOP = 1024
_TR = 128


def _orderable_i32(x):
    b = lax.bitcast_convert_type(x, jnp.int32)
    mask = lax.shift_right_arithmetic(b, 31) & jnp.int32(0x7FFFFFFF)
    return b ^ mask


def _fused_body(x_ref, wq_ref, bq_ref, wk_ref, bk_ref, wv_ref, bv_ref,
                wg1_ref, bg1_ref, wg2_ref, bg2_ref, edge_ref, wo_ref, bo_ref,
                out_ref, mask_ref, qs, ks, vs, cols):
    rr = pl.program_id(1)

    @pl.when(rr == 0)
    def _proj():
        x = x_ref[0]
        dn = (((1,), (1,)), ((), ()))
        qs[...] = (lax.dot_general(x, wq_ref[...], dn,
                                   preferred_element_type=jnp.float32)
                   + bq_ref[...]).astype(jnp.bfloat16)
        ks[...] = (lax.dot_general(x, wk_ref[...], dn,
                                   preferred_element_type=jnp.float32)
                   + bk_ref[...]).astype(jnp.bfloat16)
        vs[...] = (lax.dot_general(x, wv_ref[...], dn,
                                   preferred_element_type=jnp.float32)
                   + bv_ref[...]).astype(jnp.bfloat16)
        h1 = jax.nn.relu(lax.dot_general(x, wg1_ref[...], dn,
                                         preferred_element_type=jnp.float32)
                         + bg1_ref[...])
        scores = lax.dot_general(wg2_ref[...], h1, (((1,), (1,)), ((), ())),
                                 preferred_element_type=jnp.float32) + bg2_ref[...]
        skey = _orderable_i32(scores)

        def bit_step(i, t_u):
            bit = lax.shift_left(jnp.int32(1), jnp.int32(31) - i)
            cand_u = t_u | bit
            cand_s = cand_u ^ jnp.int32(-2147483648)
            cnt = jnp.sum((skey >= cand_s).astype(jnp.int32))
            return lax.select(cnt >= _KTOP, cand_u, t_u)

        t_u = lax.fori_loop(0, 32, bit_step, jnp.int32(0))
        t_s = t_u ^ jnp.int32(-2147483648)
        gt = skey > t_s
        eq = skey == t_s
        c = jnp.sum(gt.astype(jnp.int32))
        m = _KTOP - c

        idx = lax.broadcasted_iota(jnp.int32, (1, _N), 1)

        def idx_step(_, lohi):
            lo, hi = lohi
            mid = (lo + hi) // 2
            cnt = jnp.sum((eq & (idx < mid)).astype(jnp.int32))
            return (lax.select(cnt >= m, lo, mid + 1),
                    lax.select(cnt >= m, mid, hi))

        lo, hi = lax.fori_loop(0, 12, idx_step, (jnp.int32(0), jnp.int32(_N)))
        sel = gt | (eq & (idx < lo))
        cols[...] = sel.astype(jnp.float32)

    @pl.when(rr > 0)
    def _attn():
        base = (rr - 1) * _TR
        mask_t = jnp.maximum(edge_ref[...], cols[...])
        for h in range(_H):
            mask_ref[0, h] = mask_t
        q = qs[pl.ds(base, _TR), :]
        outs = []
        for h in range(_H):
            qh = q[:, h * _HD:(h + 1) * _HD]
            kh = ks[:, h * _HD:(h + 1) * _HD]
            vh = vs[:, h * _HD:(h + 1) * _HD]
            s = lax.dot_general(qh, kh, (((1,), (1,)), ((), ())),
                                preferred_element_type=jnp.float32) * 0.125
            mx = jnp.max(s, axis=1, keepdims=True)
            e = jnp.exp(s - mx)
            z = jnp.sum(e, axis=1, keepdims=True)
            me = e * mask_t
            sm = jnp.sum(me, axis=1, keepdims=True)
            p = (me / (sm + 1e-8 * z)).astype(jnp.bfloat16)
            outs.append(lax.dot_general(p, vh, (((1,), (0,)), ((), ())),
                                        preferred_element_type=jnp.float32))
        o = jnp.concatenate(outs, axis=1)
        out_ref[0] = lax.dot_general(o, wo_ref[...], (((1,), (1,)), ((), ())),
                                     preferred_element_type=jnp.float32) + bo_ref[...]


_E = 32768
_SEG = 8192
_RROWS = 32
_NREG = _N // _RROWS


def _edge_scatter_body(rows_hbm, cols_hbm, out_hbm, vbuf,
                       rbuf0, rbuf1, cbuf0, cbuf1, shr, shc,
                       sem_r0, sem_r1, sem_c0, sem_c1, sem_o):
    ncores = 2
    sid = lax.axis_index("s")
    wid = sid * ncores + lax.axis_index("c")
    zeros16 = jnp.zeros((16,), jnp.float32)
    ones16 = jnp.full((16,), 1.0, jnp.float32)
    nseg = _E // _SEG
    rbufs, cbufs = (rbuf0, rbuf1), (cbuf0, cbuf1)
    sems_r, sems_c = (sem_r0, sem_r1), (sem_c0, sem_c1)

    @pl.when(sid == 0)
    def _stage():
        pltpu.sync_copy(rows_hbm, shr)
        pltpu.sync_copy(cols_hbm, shc)

    plsc.subcore_barrier()

    def start_seg(s):
        b = s % 2
        return (pltpu.async_copy(shr.at[pl.ds(s * _SEG, _SEG)],
                                 rbufs[b], sems_r[b]),
                pltpu.async_copy(shc.at[pl.ds(s * _SEG, _SEG)],
                                 cbufs[b], sems_c[b]))

    pend = {0: start_seg(0)}
    out_h = None
    for p in range(2):
        reg = wid + 32 * p
        lo = reg * _RROWS
        if out_h is not None:
            out_h.wait()

        def zinit(i, _):
            vbuf[pl.ds(pl.multiple_of(i * 16, 16), 16)] = zeros16
            return 0

        lax.fori_loop(0, _RROWS * _N // 16, zinit, 0, unroll=8)
        for s in range(nseg):
            b = s % 2
            nxt = s + 1 if s + 1 < nseg else (0 if p == 0 else None)
            if nxt is not None and nxt not in pend:
                pend[nxt] = start_seg(nxt)
            hr, hc = pend.pop(s)
            hr.wait()
            hc.wait()
            rbuf, cbuf = rbufs[b], cbufs[b]

            def scan(i, _):
                off = pl.ds(pl.multiple_of(i * 16, 16), 16)
                r = rbuf[off]
                c = cbuf[off]
                sel = (r >= lo) & (r < lo + _RROWS)
                rl = jnp.where(sel, r - lo, 0)
                plsc.store_scatter(vbuf, [rl * _N + c], ones16, mask=sel)
                return 0

            lax.fori_loop(0, _SEG // 16, scan, 0, unroll=8)
        out_h = pltpu.async_copy(vbuf, out_hbm.at[pl.ds(lo * _N, _RROWS * _N)],
                                 sem_o)
    out_h.wait()


def _edge_mask_sc(edge_index):
    n = _N
    body = functools.partial(
        pl.kernel,
        out_type=jax.ShapeDtypeStruct((n * n,), jnp.float32),
        mesh=plsc.VectorSubcoreMesh(core_axis_name="c", subcore_axis_name="s"),
        compiler_params=pltpu.CompilerParams(needs_layout_passes=False),
        scratch_types=[
            pltpu.VMEM((_RROWS * n,), jnp.float32),
            pltpu.VMEM((_SEG,), jnp.int32),
            pltpu.VMEM((_SEG,), jnp.int32),
            pltpu.VMEM((_SEG,), jnp.int32),
            pltpu.VMEM((_SEG,), jnp.int32),
            pltpu.VMEM_SHARED((_E,), jnp.int32),
            pltpu.VMEM_SHARED((_E,), jnp.int32),
            pltpu.SemaphoreType.DMA,
            pltpu.SemaphoreType.DMA,
            pltpu.SemaphoreType.DMA,
            pltpu.SemaphoreType.DMA,
            pltpu.SemaphoreType.DMA,
        ],
    )(_edge_scatter_body)
    flat = body(edge_index[0], edge_index[1])
    return flat.reshape(n, n)


def kernel(x, Wq, bq, Wk, bk, Wv, bv, Wo, bo, Wg1, bg1, Wg2, bg2, edge_index):
    b, n, d = x.shape
    f32 = jnp.float32

    edge_mask = _edge_mask_sc(edge_index)

    nr = n // _TR
    out, sparse_mask = pl.pallas_call(
        _fused_body,
        grid=(b, nr + 1),
        in_specs=[
            pl.BlockSpec((1, n, d), lambda i, r: (i, 0, 0)),
            pl.BlockSpec((d, d), lambda i, r: (0, 0)),
            pl.BlockSpec((1, d), lambda i, r: (0, 0)),
            pl.BlockSpec((d, d), lambda i, r: (0, 0)),
            pl.BlockSpec((1, d), lambda i, r: (0, 0)),
            pl.BlockSpec((d, d), lambda i, r: (0, 0)),
            pl.BlockSpec((1, d), lambda i, r: (0, 0)),
            pl.BlockSpec((d // 2, d), lambda i, r: (0, 0)),
            pl.BlockSpec((1, d // 2), lambda i, r: (0, 0)),
            pl.BlockSpec((1, d // 2), lambda i, r: (0, 0)),
            pl.BlockSpec((1, n), lambda i, r: (0, 0)),
            pl.BlockSpec((_TR, n), lambda i, r: (jnp.maximum(r - 1, 0), 0)),
            pl.BlockSpec((d, d), lambda i, r: (0, 0)),
            pl.BlockSpec((1, d), lambda i, r: (0, 0)),
        ],
        out_specs=[
            pl.BlockSpec((1, _TR, d), lambda i, r: (i, jnp.maximum(r - 1, 0), 0)),
            pl.BlockSpec((1, _H, _TR, n),
                         lambda i, r: (i, 0, jnp.maximum(r - 1, 0), 0)),
        ],
        out_shape=[
            jax.ShapeDtypeStruct((b, n, d), f32),
            jax.ShapeDtypeStruct((b, _H, n, n), f32),
        ],
        scratch_shapes=[
            pltpu.VMEM((n, d), jnp.bfloat16),
            pltpu.VMEM((n, d), jnp.bfloat16),
            pltpu.VMEM((n, d), jnp.bfloat16),
            pltpu.VMEM((1, n), f32),
        ],
        compiler_params=pltpu.CompilerParams(
            vmem_limit_bytes=50 * 1024 * 1024),
    )(x, Wq, bq.reshape(1, d), Wk, bk.reshape(1, d), Wv, bv.reshape(1, d),
      Wg1, bg1.reshape(1, d // 2), Wg2,
      jnp.broadcast_to(bg2.reshape(1, 1), (1, n)),
      edge_mask, Wo, bo.reshape(1, d))

    return out, sparse_mask


# speedup vs baseline: 1.0726x; 1.0726x over previous
import functools

import jax
import jax.numpy as jnp
from jax import lax
from jax.experimental import pallas as pl
from jax.experimental.pallas import tpu as pltpu
from jax.experimental.pallas import tpu_sc as plsc

_B, _N, _D, _H = 2, 2048, 256, 4
_HD = _D // _H
_KTOP = 1024
_TR = 128


def _orderable_i32(x):
    b = lax.bitcast_convert_type(x, jnp.int32)
    mask = lax.shift_right_arithmetic(b, 31) & jnp.int32(0x7FFFFFFF)
    return b ^ mask


def _proj_body(x_ref, wq_ref, bq_ref, wk_ref, bk_ref, wv_ref, bv_ref,
               wg1_ref, bg1_ref, wg2_ref, bg2_ref,
               q_ref, k_ref, v_ref, col_ref):
    x = x_ref[0]
    dn = (((1,), (1,)), ((), ()))
    q_ref[0] = (lax.dot_general(x, wq_ref[...], dn,
                                preferred_element_type=jnp.float32)
                + bq_ref[...]).astype(jnp.bfloat16)
    k_ref[0] = (lax.dot_general(x, wk_ref[...], dn,
                                preferred_element_type=jnp.float32)
                + bk_ref[...]).astype(jnp.bfloat16)
    v_ref[0] = (lax.dot_general(x, wv_ref[...], dn,
                                preferred_element_type=jnp.float32)
                + bv_ref[...]).astype(jnp.bfloat16)
    h1 = jax.nn.relu(lax.dot_general(x, wg1_ref[...], dn,
                                     preferred_element_type=jnp.float32)
                     + bg1_ref[...])
    scores = lax.dot_general(wg2_ref[...], h1, (((1,), (1,)), ((), ())),
                             preferred_element_type=jnp.float32) + bg2_ref[...]
    skey = _orderable_i32(scores)

    def bit_step(i, t_u):
        bit = lax.shift_left(jnp.int32(1), jnp.int32(31) - i)
        cand_u = t_u | bit
        cand_s = cand_u ^ jnp.int32(-2147483648)
        cnt = jnp.sum((skey >= cand_s).astype(jnp.int32))
        return lax.select(cnt >= _KTOP, cand_u, t_u)

    t_u = lax.fori_loop(0, 32, bit_step, jnp.int32(0))
    t_s = t_u ^ jnp.int32(-2147483648)
    gt = skey > t_s
    eq = skey == t_s
    c = jnp.sum(gt.astype(jnp.int32))
    m = _KTOP - c

    idx = lax.broadcasted_iota(jnp.int32, (1, _N), 1)

    def idx_step(_, lohi):
        lo, hi = lohi
        mid = (lo + hi) // 2
        cnt = jnp.sum((eq & (idx < mid)).astype(jnp.int32))
        return (lax.select(cnt >= m, lo, mid + 1),
                lax.select(cnt >= m, mid, hi))

    lo, hi = lax.fori_loop(0, 12, idx_step, (jnp.int32(0), jnp.int32(_N)))
    sel = gt | (eq & (idx < lo))
    col_ref[0] = sel.astype(jnp.float32)


def _attn_body(q_ref, k_ref, v_ref, edge_ref, col_ref, wo_ref, bo_ref,
               out_ref, mask_ref):
    mask_t = jnp.maximum(edge_ref[...], col_ref[0])
    for h in range(_H):
        mask_ref[0, h] = mask_t
    q = q_ref[0]
    outs = []
    for h in range(_H):
        qh = q[:, h * _HD:(h + 1) * _HD]
        kh = k_ref[0][:, h * _HD:(h + 1) * _HD]
        vh = v_ref[0][:, h * _HD:(h + 1) * _HD]
        s = lax.dot_general(qh, kh, (((1,), (1,)), ((), ())),
                            preferred_element_type=jnp.float32) * 0.125
        mx = jnp.max(s, axis=1, keepdims=True)
        e = jnp.exp(s - mx)
        z = jnp.sum(e, axis=1, keepdims=True)
        me = e * mask_t
        sm = jnp.sum(me, axis=1, keepdims=True)
        p = (me / (sm + 1e-8 * z)).astype(jnp.bfloat16)
        outs.append(lax.dot_general(p, vh, (((1,), (0,)), ((), ())),
                                    preferred_element_type=jnp.float32))
    o = jnp.concatenate(outs, axis=1)
    out_ref[0] = lax.dot_general(o, wo_ref[...], (((1,), (1,)), ((), ())),
                                 preferred_element_type=jnp.float32) + bo_ref[...]


_E = 32768
_SEG = 8192
_RROWS = 32
_NREG = _N // _RROWS


def _edge_scatter_body(rows_hbm, cols_hbm, out_hbm, vbuf,
                       rbuf0, rbuf1, cbuf0, cbuf1, shr, shc,
                       sem_r0, sem_r1, sem_c0, sem_c1, sem_o):
    ncores = 2
    sid = lax.axis_index("s")
    wid = sid * ncores + lax.axis_index("c")
    zeros16 = jnp.zeros((16,), jnp.float32)
    ones16 = jnp.full((16,), 1.0, jnp.float32)
    nseg = _E // _SEG
    rbufs, cbufs = (rbuf0, rbuf1), (cbuf0, cbuf1)
    sems_r, sems_c = (sem_r0, sem_r1), (sem_c0, sem_c1)

    @pl.when(sid == 0)
    def _stage():
        pltpu.sync_copy(rows_hbm, shr)
        pltpu.sync_copy(cols_hbm, shc)

    plsc.subcore_barrier()

    def start_seg(s):
        b = s % 2
        return (pltpu.async_copy(shr.at[pl.ds(s * _SEG, _SEG)],
                                 rbufs[b], sems_r[b]),
                pltpu.async_copy(shc.at[pl.ds(s * _SEG, _SEG)],
                                 cbufs[b], sems_c[b]))

    pend = {0: start_seg(0)}
    out_h = None
    for p in range(2):
        reg = wid + 32 * p
        lo = reg * _RROWS
        if out_h is not None:
            out_h.wait()

        def zinit(i, _):
            vbuf[pl.ds(pl.multiple_of(i * 16, 16), 16)] = zeros16
            return 0

        lax.fori_loop(0, _RROWS * _N // 16, zinit, 0, unroll=8)
        for s in range(nseg):
            b = s % 2
            nxt = s + 1 if s + 1 < nseg else (0 if p == 0 else None)
            if nxt is not None and nxt not in pend:
                pend[nxt] = start_seg(nxt)
            hr, hc = pend.pop(s)
            hr.wait()
            hc.wait()
            rbuf, cbuf = rbufs[b], cbufs[b]

            def scan(i, _):
                off = pl.ds(pl.multiple_of(i * 16, 16), 16)
                r = rbuf[off]
                c = cbuf[off]
                sel = (r >= lo) & (r < lo + _RROWS)
                rl = jnp.where(sel, r - lo, 0)
                plsc.store_scatter(vbuf, [rl * _N + c], ones16, mask=sel)
                return 0

            lax.fori_loop(0, _SEG // 16, scan, 0, unroll=8)
        out_h = pltpu.async_copy(vbuf, out_hbm.at[pl.ds(lo * _N, _RROWS * _N)],
                                 sem_o)
    out_h.wait()


def _edge_mask_sc(edge_index):
    n = _N
    body = functools.partial(
        pl.kernel,
        out_type=jax.ShapeDtypeStruct((n * n,), jnp.float32),
        mesh=plsc.VectorSubcoreMesh(core_axis_name="c", subcore_axis_name="s"),
        compiler_params=pltpu.CompilerParams(needs_layout_passes=False),
        scratch_types=[
            pltpu.VMEM((_RROWS * n,), jnp.float32),
            pltpu.VMEM((_SEG,), jnp.int32),
            pltpu.VMEM((_SEG,), jnp.int32),
            pltpu.VMEM((_SEG,), jnp.int32),
            pltpu.VMEM((_SEG,), jnp.int32),
            pltpu.VMEM_SHARED((_E,), jnp.int32),
            pltpu.VMEM_SHARED((_E,), jnp.int32),
            pltpu.SemaphoreType.DMA,
            pltpu.SemaphoreType.DMA,
            pltpu.SemaphoreType.DMA,
            pltpu.SemaphoreType.DMA,
            pltpu.SemaphoreType.DMA,
        ],
    )(_edge_scatter_body)
    flat = body(edge_index[0], edge_index[1])
    return flat.reshape(n, n)


def kernel(x, Wq, bq, Wk, bk, Wv, bv, Wo, bo, Wg1, bg1, Wg2, bg2, edge_index):
    b, n, d = x.shape
    f32 = jnp.float32
    bf16 = jnp.bfloat16

    q, k, v, col_mask = pl.pallas_call(
        _proj_body,
        grid=(b,),
        in_specs=[
            pl.BlockSpec((1, n, d), lambda i: (i, 0, 0)),
            pl.BlockSpec((d, d), lambda i: (0, 0)),
            pl.BlockSpec((1, d), lambda i: (0, 0)),
            pl.BlockSpec((d, d), lambda i: (0, 0)),
            pl.BlockSpec((1, d), lambda i: (0, 0)),
            pl.BlockSpec((d, d), lambda i: (0, 0)),
            pl.BlockSpec((1, d), lambda i: (0, 0)),
            pl.BlockSpec((d // 2, d), lambda i: (0, 0)),
            pl.BlockSpec((1, d // 2), lambda i: (0, 0)),
            pl.BlockSpec((1, d // 2), lambda i: (0, 0)),
            pl.BlockSpec((1, n), lambda i: (0, 0)),
        ],
        out_specs=[
            pl.BlockSpec((1, n, d), lambda i: (i, 0, 0)),
            pl.BlockSpec((1, n, d), lambda i: (i, 0, 0)),
            pl.BlockSpec((1, n, d), lambda i: (i, 0, 0)),
            pl.BlockSpec((1, 1, n), lambda i: (i, 0, 0)),
        ],
        out_shape=[
            jax.ShapeDtypeStruct((b, n, d), bf16),
            jax.ShapeDtypeStruct((b, n, d), bf16),
            jax.ShapeDtypeStruct((b, n, d), bf16),
            jax.ShapeDtypeStruct((b, 1, n), f32),
        ],
    )(x, Wq, bq.reshape(1, d), Wk, bk.reshape(1, d), Wv, bv.reshape(1, d),
      Wg1, bg1.reshape(1, d // 2), Wg2,
      jnp.broadcast_to(bg2.reshape(1, 1), (1, n)))

    edge_mask = _edge_mask_sc(edge_index)

    nr = n // _TR
    out, sparse_mask = pl.pallas_call(
        _attn_body,
        grid=(b, nr),
        in_specs=[
            pl.BlockSpec((1, _TR, d), lambda i, r: (i, r, 0)),
            pl.BlockSpec((1, n, d), lambda i, r: (i, 0, 0)),
            pl.BlockSpec((1, n, d), lambda i, r: (i, 0, 0)),
            pl.BlockSpec((_TR, n), lambda i, r: (r, 0)),
            pl.BlockSpec((1, 1, n), lambda i, r: (i, 0, 0)),
            pl.BlockSpec((d, d), lambda i, r: (0, 0)),
            pl.BlockSpec((1, d), lambda i, r: (0, 0)),
        ],
        out_specs=[
            pl.BlockSpec((1, _TR, d), lambda i, r: (i, r, 0)),
            pl.BlockSpec((1, _H, _TR, n), lambda i, r: (i, 0, r, 0)),
        ],
        out_shape=[
            jax.ShapeDtypeStruct((b, n, d), f32),
            jax.ShapeDtypeStruct((b, _H, n, n), f32),
        ],
    )(q, k, v, edge_mask, col_mask, Wo, bo.reshape(1, d))

    return out, sparse_mask


# speedup vs baseline: 1.1077x; 1.0328x over previous
import functools

import jax
import jax.numpy as jnp
from jax import lax
from jax.experimental import pallas as pl
from jax.experimental.pallas import tpu as pltpu
from jax.experimental.pallas import tpu_sc as plsc

_B, _N, _D, _H = 2, 2048, 256, 4
_HD = _D // _H
_KTOP = 1024
_TR = 128


def _orderable_i32(x):
    b = lax.bitcast_convert_type(x, jnp.int32)
    mask = lax.shift_right_arithmetic(b, 31) & jnp.int32(0x7FFFFFFF)
    return b ^ mask


def _proj_body(x_ref, wq_ref, bq_ref, wk_ref, bk_ref, wv_ref, bv_ref,
               wg1_ref, bg1_ref, wg2_ref, bg2_ref,
               q_ref, k_ref, v_ref, col_ref):
    x = x_ref[0]
    dn = (((1,), (1,)), ((), ()))
    q_ref[0] = ((lax.dot_general(x, wq_ref[...], dn,
                                 preferred_element_type=jnp.float32)
                 + bq_ref[...]) * 0.125).astype(jnp.bfloat16)
    k_ref[0] = (lax.dot_general(x, wk_ref[...], dn,
                                preferred_element_type=jnp.float32)
                + bk_ref[...]).astype(jnp.bfloat16)
    v_ref[0] = (lax.dot_general(x, wv_ref[...], dn,
                                preferred_element_type=jnp.float32)
                + bv_ref[...]).astype(jnp.bfloat16)
    h1 = jax.nn.relu(lax.dot_general(x, wg1_ref[...], dn,
                                     preferred_element_type=jnp.float32)
                     + bg1_ref[...])
    scores = lax.dot_general(wg2_ref[...], h1, (((1,), (1,)), ((), ())),
                             preferred_element_type=jnp.float32) + bg2_ref[...]
    skey = _orderable_i32(scores)

    def bit_step(i, t_u):
        bit = lax.shift_left(jnp.int32(1), jnp.int32(31) - i)
        cand_u = t_u | bit
        cand_s = cand_u ^ jnp.int32(-2147483648)
        cnt = jnp.sum((skey >= cand_s).astype(jnp.int32))
        return lax.select(cnt >= _KTOP, cand_u, t_u)

    t_u = lax.fori_loop(0, 32, bit_step, jnp.int32(0))
    t_s = t_u ^ jnp.int32(-2147483648)
    gt = skey > t_s
    eq = skey == t_s
    c = jnp.sum(gt.astype(jnp.int32))
    m = _KTOP - c

    idx = lax.broadcasted_iota(jnp.int32, (1, _N), 1)

    def idx_step(_, lohi):
        lo, hi = lohi
        mid = (lo + hi) // 2
        cnt = jnp.sum((eq & (idx < mid)).astype(jnp.int32))
        return (lax.select(cnt >= m, lo, mid + 1),
                lax.select(cnt >= m, mid, hi))

    lo, hi = lax.fori_loop(0, 12, idx_step, (jnp.int32(0), jnp.int32(_N)))
    sel = gt | (eq & (idx < lo))
    col_ref[0] = sel.astype(jnp.float32)


def _attn_body(q_ref, k_ref, v_ref, edge_ref, col_ref, wo_ref, bo_ref,
               out_ref, mask_ref):
    mask_t = jnp.maximum(edge_ref[...], col_ref[0])
    for h in range(_H):
        mask_ref[0, h] = mask_t
    q = q_ref[0]
    outs = []
    for h in range(_H):
        qh = q[:, h * _HD:(h + 1) * _HD]
        kh = k_ref[0][:, h * _HD:(h + 1) * _HD]
        vh = v_ref[0][:, h * _HD:(h + 1) * _HD]
        s = lax.dot_general(qh, kh, (((1,), (1,)), ((), ())),
                            preferred_element_type=jnp.float32)
        mx = jnp.max(s, axis=1, keepdims=True)
        e = jnp.exp(s - mx)
        z = jnp.sum(e, axis=1, keepdims=True)
        me = e * mask_t
        sm = jnp.sum(me, axis=1, keepdims=True)
        p = (me / (sm + 1e-8 * z)).astype(jnp.bfloat16)
        outs.append(lax.dot_general(p, vh, (((1,), (0,)), ((), ())),
                                    preferred_element_type=jnp.float32))
    o = jnp.concatenate(outs, axis=1)
    out_ref[0] = lax.dot_general(o, wo_ref[...], (((1,), (1,)), ((), ())),
                                 preferred_element_type=jnp.float32) + bo_ref[...]


_E = 32768
_SEG = 8192
_RROWS = 32
_NREG = _N // _RROWS


def _edge_scatter_body(rows_hbm, cols_hbm, out_hbm, vbuf,
                       rbuf0, rbuf1, cbuf0, cbuf1, shr, shc,
                       sem_r0, sem_r1, sem_c0, sem_c1, sem_o):
    ncores = 2
    sid = lax.axis_index("s")
    wid = sid * ncores + lax.axis_index("c")
    zeros16 = jnp.zeros((16,), jnp.float32)
    ones16 = jnp.full((16,), 1.0, jnp.float32)
    nseg = _E // _SEG
    rbufs, cbufs = (rbuf0, rbuf1), (cbuf0, cbuf1)
    sems_r, sems_c = (sem_r0, sem_r1), (sem_c0, sem_c1)

    @pl.when(sid == 0)
    def _stage():
        pltpu.sync_copy(rows_hbm, shr)
        pltpu.sync_copy(cols_hbm, shc)

    plsc.subcore_barrier()

    def start_seg(s):
        b = s % 2
        return (pltpu.async_copy(shr.at[pl.ds(s * _SEG, _SEG)],
                                 rbufs[b], sems_r[b]),
                pltpu.async_copy(shc.at[pl.ds(s * _SEG, _SEG)],
                                 cbufs[b], sems_c[b]))

    pend = {0: start_seg(0)}
    out_h = None
    for p in range(2):
        reg = wid + 32 * p
        lo = reg * _RROWS
        if out_h is not None:
            out_h.wait()

        for zr in range(_RROWS):
            def zinit(i, _):
                vbuf[zr, pl.ds(pl.multiple_of(i * 16, 16), 16)] = zeros16
                return 0

            lax.fori_loop(0, _N // 16, zinit, 0, unroll=8)
        for s in range(nseg):
            b = s % 2
            nxt = s + 1 if s + 1 < nseg else (0 if p == 0 else None)
            if nxt is not None and nxt not in pend:
                pend[nxt] = start_seg(nxt)
            hr, hc = pend.pop(s)
            hr.wait()
            hc.wait()
            rbuf, cbuf = rbufs[b], cbufs[b]

            def scan(i, _):
                off = pl.ds(pl.multiple_of(i * 16, 16), 16)
                r = rbuf[off]
                c = cbuf[off]
                sel = (r >= lo) & (r < lo + _RROWS)
                rl = jnp.where(sel, r - lo, 0)
                plsc.store_scatter(vbuf, [rl, c], ones16, mask=sel)
                return 0

            lax.fori_loop(0, _SEG // 16, scan, 0, unroll=8)
        out_h = pltpu.async_copy(vbuf, out_hbm.at[pl.ds(lo, _RROWS), :],
                                 sem_o)
    out_h.wait()


def _edge_mask_sc(edge_index):
    n = _N
    body = functools.partial(
        pl.kernel,
        out_type=jax.ShapeDtypeStruct((n, n), jnp.float32),
        mesh=plsc.VectorSubcoreMesh(core_axis_name="c", subcore_axis_name="s"),
        compiler_params=pltpu.CompilerParams(needs_layout_passes=False),
        scratch_types=[
            pltpu.VMEM((_RROWS, n), jnp.float32),
            pltpu.VMEM((_SEG,), jnp.int32),
            pltpu.VMEM((_SEG,), jnp.int32),
            pltpu.VMEM((_SEG,), jnp.int32),
            pltpu.VMEM((_SEG,), jnp.int32),
            pltpu.VMEM_SHARED((_E,), jnp.int32),
            pltpu.VMEM_SHARED((_E,), jnp.int32),
            pltpu.SemaphoreType.DMA,
            pltpu.SemaphoreType.DMA,
            pltpu.SemaphoreType.DMA,
            pltpu.SemaphoreType.DMA,
            pltpu.SemaphoreType.DMA,
        ],
    )(_edge_scatter_body)
    return body(edge_index[0], edge_index[1])


def kernel(x, Wq, bq, Wk, bk, Wv, bv, Wo, bo, Wg1, bg1, Wg2, bg2, edge_index):
    b, n, d = x.shape
    f32 = jnp.float32
    bf16 = jnp.bfloat16

    q, k, v, col_mask = pl.pallas_call(
        _proj_body,
        grid=(b,),
        in_specs=[
            pl.BlockSpec((1, n, d), lambda i: (i, 0, 0)),
            pl.BlockSpec((d, d), lambda i: (0, 0)),
            pl.BlockSpec((1, d), lambda i: (0, 0)),
            pl.BlockSpec((d, d), lambda i: (0, 0)),
            pl.BlockSpec((1, d), lambda i: (0, 0)),
            pl.BlockSpec((d, d), lambda i: (0, 0)),
            pl.BlockSpec((1, d), lambda i: (0, 0)),
            pl.BlockSpec((d // 2, d), lambda i: (0, 0)),
            pl.BlockSpec((1, d // 2), lambda i: (0, 0)),
            pl.BlockSpec((1, d // 2), lambda i: (0, 0)),
            pl.BlockSpec((1, n), lambda i: (0, 0)),
        ],
        out_specs=[
            pl.BlockSpec((1, n, d), lambda i: (i, 0, 0)),
            pl.BlockSpec((1, n, d), lambda i: (i, 0, 0)),
            pl.BlockSpec((1, n, d), lambda i: (i, 0, 0)),
            pl.BlockSpec((1, 1, n), lambda i: (i, 0, 0)),
        ],
        out_shape=[
            jax.ShapeDtypeStruct((b, n, d), bf16),
            jax.ShapeDtypeStruct((b, n, d), bf16),
            jax.ShapeDtypeStruct((b, n, d), bf16),
            jax.ShapeDtypeStruct((b, 1, n), f32),
        ],
    )(x, Wq, bq.reshape(1, d), Wk, bk.reshape(1, d), Wv, bv.reshape(1, d),
      Wg1, bg1.reshape(1, d // 2), Wg2,
      jnp.broadcast_to(bg2.reshape(1, 1), (1, n)))

    edge_mask = _edge_mask_sc(edge_index)

    nr = n // _TR
    out, sparse_mask = pl.pallas_call(
        _attn_body,
        grid=(b, nr),
        in_specs=[
            pl.BlockSpec((1, _TR, d), lambda i, r: (i, r, 0)),
            pl.BlockSpec((1, n, d), lambda i, r: (i, 0, 0)),
            pl.BlockSpec((1, n, d), lambda i, r: (i, 0, 0)),
            pl.BlockSpec((_TR, n), lambda i, r: (r, 0)),
            pl.BlockSpec((1, 1, n), lambda i, r: (i, 0, 0)),
            pl.BlockSpec((d, d), lambda i, r: (0, 0)),
            pl.BlockSpec((1, d), lambda i, r: (0, 0)),
        ],
        out_specs=[
            pl.BlockSpec((1, _TR, d), lambda i, r: (i, r, 0)),
            pl.BlockSpec((1, _H, _TR, n), lambda i, r: (i, 0, r, 0)),
        ],
        out_shape=[
            jax.ShapeDtypeStruct((b, n, d), f32),
            jax.ShapeDtypeStruct((b, _H, n, n), f32),
        ],
    )(q, k, v, edge_mask, col_mask, Wo, bo.reshape(1, d))

    return out, sparse_mask


# speedup vs baseline: 1.3018x; 1.1752x over previous
import functools

import jax
import jax.numpy as jnp
from jax import lax
from jax.experimental import pallas as pl
from jax.experimental.pallas import tpu as pltpu
from jax.experimental.pallas import tpu_sc as plsc

_B, _N, _D, _H = 2, 2048, 256, 4
_HD = _D // _H
_KTOP = 1024
_TR = 256


def _orderable_i32(x):
    b = lax.bitcast_convert_type(x, jnp.int32)
    mask = lax.shift_right_arithmetic(b, 31) & jnp.int32(0x7FFFFFFF)
    return b ^ mask


def _proj_body(x_ref, wq_ref, bq_ref, wk_ref, bk_ref, wv_ref, bv_ref,
               wg1_ref, bg1_ref, wg2_ref, bg2_ref,
               q_ref, k_ref, v_ref, col_ref):
    x = x_ref[0]
    dn = (((1,), (1,)), ((), ()))
    q_ref[0] = ((lax.dot_general(x, wq_ref[...], dn,
                                 preferred_element_type=jnp.float32)
                 + bq_ref[...]) * 0.125).astype(jnp.bfloat16)
    k_ref[0] = (lax.dot_general(x, wk_ref[...], dn,
                                preferred_element_type=jnp.float32)
                + bk_ref[...]).astype(jnp.bfloat16)
    v_ref[0] = (lax.dot_general(x, wv_ref[...], dn,
                                preferred_element_type=jnp.float32)
                + bv_ref[...]).astype(jnp.bfloat16)
    h1 = jax.nn.relu(lax.dot_general(x, wg1_ref[...], dn,
                                     preferred_element_type=jnp.float32)
                     + bg1_ref[...])
    scores = lax.dot_general(wg2_ref[...], h1, (((1,), (1,)), ((), ())),
                             preferred_element_type=jnp.float32) + bg2_ref[...]
    skey = _orderable_i32(scores)

    def bit_step(i, t_u):
        bit = lax.shift_left(jnp.int32(1), jnp.int32(31) - i)
        cand_u = t_u | bit
        cand_s = cand_u ^ jnp.int32(-2147483648)
        cnt = jnp.sum((skey >= cand_s).astype(jnp.int32))
        return lax.select(cnt >= _KTOP, cand_u, t_u)

    t_u = lax.fori_loop(0, 32, bit_step, jnp.int32(0))
    t_s = t_u ^ jnp.int32(-2147483648)
    gt = skey > t_s
    eq = skey == t_s
    c = jnp.sum(gt.astype(jnp.int32))
    m = _KTOP - c

    idx = lax.broadcasted_iota(jnp.int32, (1, _N), 1)

    def idx_step(_, lohi):
        lo, hi = lohi
        mid = (lo + hi) // 2
        cnt = jnp.sum((eq & (idx < mid)).astype(jnp.int32))
        return (lax.select(cnt >= m, lo, mid + 1),
                lax.select(cnt >= m, mid, hi))

    lo, hi = lax.fori_loop(0, 12, idx_step, (jnp.int32(0), jnp.int32(_N)))
    sel = gt | (eq & (idx < lo))
    col_ref[0] = sel.astype(jnp.float32)


def _attn_body(q_ref, k_ref, v_ref, edge_ref, col_ref, wo_ref, bo_ref,
               out_ref, mask_ref):
    mask_t = jnp.maximum(edge_ref[...], col_ref[0])
    for h in range(_H):
        mask_ref[0, h] = mask_t
    m2 = mask_t + 1e-8
    q = q_ref[0]
    outs = []
    for h in range(_H):
        qh = q[:, h * _HD:(h + 1) * _HD]
        kh = k_ref[0][:, h * _HD:(h + 1) * _HD]
        vh = v_ref[0][:, h * _HD:(h + 1) * _HD]
        s = lax.dot_general(qh, kh, (((1,), (1,)), ((), ())),
                            preferred_element_type=jnp.float32)
        mx = jnp.max(s, axis=1, keepdims=True)
        e = jnp.exp(s - mx)
        me = e * m2
        sm = jnp.sum(me, axis=1, keepdims=True)
        p = (me / sm).astype(jnp.bfloat16)
        outs.append(lax.dot_general(p, vh, (((1,), (0,)), ((), ())),
                                    preferred_element_type=jnp.float32))
    o = jnp.concatenate(outs, axis=1)
    out_ref[0] = lax.dot_general(o, wo_ref[...], (((1,), (1,)), ((), ())),
                                 preferred_element_type=jnp.float32) + bo_ref[...]


_E = 32768
_SEG = 8192
_RROWS = 32
_NREG = _N // _RROWS


def _edge_scatter_body(rows_hbm, cols_hbm, out_hbm, vbuf,
                       rbuf0, rbuf1, cbuf0, cbuf1, shr, shc,
                       sem_r0, sem_r1, sem_c0, sem_c1, sem_o):
    ncores = 2
    sid = lax.axis_index("s")
    wid = sid * ncores + lax.axis_index("c")
    zeros16 = jnp.zeros((16,), jnp.float32)
    ones16 = jnp.full((16,), 1.0, jnp.float32)
    nseg = _E // _SEG
    rbufs, cbufs = (rbuf0, rbuf1), (cbuf0, cbuf1)
    sems_r, sems_c = (sem_r0, sem_r1), (sem_c0, sem_c1)

    @pl.when(sid == 0)
    def _stage():
        pltpu.sync_copy(rows_hbm, shr)
        pltpu.sync_copy(cols_hbm, shc)

    plsc.subcore_barrier()

    def start_seg(s):
        b = s % 2
        return (pltpu.async_copy(shr.at[pl.ds(s * _SEG, _SEG)],
                                 rbufs[b], sems_r[b]),
                pltpu.async_copy(shc.at[pl.ds(s * _SEG, _SEG)],
                                 cbufs[b], sems_c[b]))

    pend = {0: start_seg(0)}
    out_h = None
    for p in range(2):
        reg = wid + 32 * p
        lo = reg * _RROWS
        if out_h is not None:
            out_h.wait()

        for zr in range(_RROWS):
            def zinit(i, _):
                vbuf[zr, pl.ds(pl.multiple_of(i * 16, 16), 16)] = zeros16
                return 0

            lax.fori_loop(0, _N // 16, zinit, 0, unroll=8)
        for s in range(nseg):
            b = s % 2
            nxt = s + 1 if s + 1 < nseg else (0 if p == 0 else None)
            if nxt is not None and nxt not in pend:
                pend[nxt] = start_seg(nxt)
            hr, hc = pend.pop(s)
            hr.wait()
            hc.wait()
            rbuf, cbuf = rbufs[b], cbufs[b]

            def scan(i, _):
                off = pl.ds(pl.multiple_of(i * 16, 16), 16)
                r = rbuf[off]
                c = cbuf[off]
                sel = (r >= lo) & (r < lo + _RROWS)
                rl = jnp.where(sel, r - lo, 0)
                plsc.store_scatter(vbuf, [rl, c], ones16, mask=sel)
                return 0

            lax.fori_loop(0, _SEG // 16, scan, 0, unroll=8)
        out_h = pltpu.async_copy(vbuf, out_hbm.at[pl.ds(lo, _RROWS), :],
                                 sem_o)
    out_h.wait()


def _edge_mask_sc(edge_index):
    n = _N
    body = functools.partial(
        pl.kernel,
        out_type=jax.ShapeDtypeStruct((n, n), jnp.float32),
        mesh=plsc.VectorSubcoreMesh(core_axis_name="c", subcore_axis_name="s"),
        compiler_params=pltpu.CompilerParams(needs_layout_passes=False),
        scratch_types=[
            pltpu.VMEM((_RROWS, n), jnp.float32),
            pltpu.VMEM((_SEG,), jnp.int32),
            pltpu.VMEM((_SEG,), jnp.int32),
            pltpu.VMEM((_SEG,), jnp.int32),
            pltpu.VMEM((_SEG,), jnp.int32),
            pltpu.VMEM_SHARED((_E,), jnp.int32),
            pltpu.VMEM_SHARED((_E,), jnp.int32),
            pltpu.SemaphoreType.DMA,
            pltpu.SemaphoreType.DMA,
            pltpu.SemaphoreType.DMA,
            pltpu.SemaphoreType.DMA,
            pltpu.SemaphoreType.DMA,
        ],
    )(_edge_scatter_body)
    return body(edge_index[0], edge_index[1])


def kernel(x, Wq, bq, Wk, bk, Wv, bv, Wo, bo, Wg1, bg1, Wg2, bg2, edge_index):
    b, n, d = x.shape
    f32 = jnp.float32
    bf16 = jnp.bfloat16

    q, k, v, col_mask = pl.pallas_call(
        _proj_body,
        grid=(b,),
        in_specs=[
            pl.BlockSpec((1, n, d), lambda i: (i, 0, 0)),
            pl.BlockSpec((d, d), lambda i: (0, 0)),
            pl.BlockSpec((1, d), lambda i: (0, 0)),
            pl.BlockSpec((d, d), lambda i: (0, 0)),
            pl.BlockSpec((1, d), lambda i: (0, 0)),
            pl.BlockSpec((d, d), lambda i: (0, 0)),
            pl.BlockSpec((1, d), lambda i: (0, 0)),
            pl.BlockSpec((d // 2, d), lambda i: (0, 0)),
            pl.BlockSpec((1, d // 2), lambda i: (0, 0)),
            pl.BlockSpec((1, d // 2), lambda i: (0, 0)),
            pl.BlockSpec((1, n), lambda i: (0, 0)),
        ],
        out_specs=[
            pl.BlockSpec((1, n, d), lambda i: (i, 0, 0)),
            pl.BlockSpec((1, n, d), lambda i: (i, 0, 0)),
            pl.BlockSpec((1, n, d), lambda i: (i, 0, 0)),
            pl.BlockSpec((1, 1, n), lambda i: (i, 0, 0)),
        ],
        out_shape=[
            jax.ShapeDtypeStruct((b, n, d), bf16),
            jax.ShapeDtypeStruct((b, n, d), bf16),
            jax.ShapeDtypeStruct((b, n, d), bf16),
            jax.ShapeDtypeStruct((b, 1, n), f32),
        ],
    )(x, Wq, bq.reshape(1, d), Wk, bk.reshape(1, d), Wv, bv.reshape(1, d),
      Wg1, bg1.reshape(1, d // 2), Wg2,
      jnp.broadcast_to(bg2.reshape(1, 1), (1, n)))

    edge_mask = _edge_mask_sc(edge_index)

    nr = n // _TR
    out, sparse_mask = pl.pallas_call(
        _attn_body,
        grid=(b, nr),
        in_specs=[
            pl.BlockSpec((1, _TR, d), lambda i, r: (i, r, 0)),
            pl.BlockSpec((1, n, d), lambda i, r: (i, 0, 0)),
            pl.BlockSpec((1, n, d), lambda i, r: (i, 0, 0)),
            pl.BlockSpec((_TR, n), lambda i, r: (r, 0)),
            pl.BlockSpec((1, 1, n), lambda i, r: (i, 0, 0)),
            pl.BlockSpec((d, d), lambda i, r: (0, 0)),
            pl.BlockSpec((1, d), lambda i, r: (0, 0)),
        ],
        out_specs=[
            pl.BlockSpec((1, _TR, d), lambda i, r: (i, r, 0)),
            pl.BlockSpec((1, _H, _TR, n), lambda i, r: (i, 0, r, 0)),
        ],
        out_shape=[
            jax.ShapeDtypeStruct((b, n, d), f32),
            jax.ShapeDtypeStruct((b, _H, n, n), f32),
        ],
        compiler_params=pltpu.CompilerParams(
            vmem_limit_bytes=100 * 1024 * 1024),
    )(q, k, v, edge_mask, col_mask, Wo, bo.reshape(1, d))

    return out, sparse_mask


# speedup vs baseline: 1.3069x; 1.0039x over previous
import functools

import jax
import jax.numpy as jnp
from jax import lax
from jax.experimental import pallas as pl
from jax.experimental.pallas import tpu as pltpu
from jax.experimental.pallas import tpu_sc as plsc

_B, _N, _D, _H = 2, 2048, 256, 4
_HD = _D // _H
_KTOP = 1024
_TR = 256


def _orderable_i32(x):
    b = lax.bitcast_convert_type(x, jnp.int32)
    mask = lax.shift_right_arithmetic(b, 31) & jnp.int32(0x7FFFFFFF)
    return b ^ mask


def _proj_body(x_ref, wq_ref, bq_ref, wk_ref, bk_ref, wv_ref, bv_ref,
               wg1_ref, bg1_ref, wg2_ref, bg2_ref,
               q_ref, k_ref, v_ref, col_ref):
    x = x_ref[0]
    dn = (((1,), (1,)), ((), ()))
    q_ref[0] = ((lax.dot_general(x, wq_ref[...], dn,
                                 preferred_element_type=jnp.float32)
                 + bq_ref[...]) * 0.125).astype(jnp.bfloat16)
    k_ref[0] = (lax.dot_general(x, wk_ref[...], dn,
                                preferred_element_type=jnp.float32)
                + bk_ref[...]).astype(jnp.bfloat16)
    v_ref[0] = (lax.dot_general(x, wv_ref[...], dn,
                                preferred_element_type=jnp.float32)
                + bv_ref[...]).astype(jnp.bfloat16)
    h1 = jax.nn.relu(lax.dot_general(x, wg1_ref[...], dn,
                                     preferred_element_type=jnp.float32)
                     + bg1_ref[...])
    scores = lax.dot_general(wg2_ref[...], h1, (((1,), (1,)), ((), ())),
                             preferred_element_type=jnp.float32) + bg2_ref[...]
    skey = _orderable_i32(scores)

    def bit_step(i, t_u):
        bit = lax.shift_left(jnp.int32(1), jnp.int32(31) - i)
        cand_u = t_u | bit
        cand_s = cand_u ^ jnp.int32(-2147483648)
        cnt = jnp.sum((skey >= cand_s).astype(jnp.int32))
        return lax.select(cnt >= _KTOP, cand_u, t_u)

    t_u = lax.fori_loop(0, 32, bit_step, jnp.int32(0))
    t_s = t_u ^ jnp.int32(-2147483648)
    gt = skey > t_s
    eq = skey == t_s
    c = jnp.sum(gt.astype(jnp.int32))
    m = _KTOP - c

    idx = lax.broadcasted_iota(jnp.int32, (1, _N), 1)

    def idx_step(_, lohi):
        lo, hi = lohi
        mid = (lo + hi) // 2
        cnt = jnp.sum((eq & (idx < mid)).astype(jnp.int32))
        return (lax.select(cnt >= m, lo, mid + 1),
                lax.select(cnt >= m, mid, hi))

    lo, hi = lax.fori_loop(0, 12, idx_step, (jnp.int32(0), jnp.int32(_N)))
    sel = gt | (eq & (idx < lo))
    col_ref[0] = sel.astype(jnp.float32)


def _attn_body(q_ref, k_ref, v_ref, edge_ref, col_ref, wo_ref, bo_ref,
               out_ref, mask_ref):
    mask_t = jnp.maximum(edge_ref[...], col_ref[0])
    for h in range(_H):
        mask_ref[0, h] = mask_t
    m2 = mask_t + 1e-8
    q = q_ref[0]
    outs = []
    for h in range(_H):
        qh = q[:, h * _HD:(h + 1) * _HD]
        kh = k_ref[0][:, h * _HD:(h + 1) * _HD]
        vh = v_ref[0][:, h * _HD:(h + 1) * _HD]
        s = lax.dot_general(qh, kh, (((1,), (1,)), ((), ())),
                            preferred_element_type=jnp.float32)
        mx = jnp.max(s, axis=1, keepdims=True)
        e = jnp.exp(s - mx)
        me = e * m2
        sm = jnp.sum(me, axis=1, keepdims=True)
        p = (me / sm).astype(jnp.bfloat16)
        outs.append(lax.dot_general(p, vh, (((1,), (0,)), ((), ())),
                                    preferred_element_type=jnp.float32))
    o = jnp.concatenate(outs, axis=1)
    out_ref[0] = lax.dot_general(o, wo_ref[...], (((1,), (1,)), ((), ())),
                                 preferred_element_type=jnp.float32) + bo_ref[...]


_E = 32768
_SEG = 8192
_RROWS = 32
_NREG = _N // _RROWS


def _edge_scatter_body(rows_hbm, cols_hbm, out_hbm, vbuf,
                       rbuf0, rbuf1, cbuf0, cbuf1, shr, shc,
                       sem_r0, sem_r1, sem_c0, sem_c1, sem_o):
    ncores = 2
    sid = lax.axis_index("s")
    wid = sid * ncores + lax.axis_index("c")
    zeros16 = jnp.zeros((16,), jnp.float32)
    ones16 = jnp.full((16,), 1.0, jnp.float32)
    nseg = _E // _SEG
    rbufs, cbufs = (rbuf0, rbuf1), (cbuf0, cbuf1)
    sems_r, sems_c = (sem_r0, sem_r1), (sem_c0, sem_c1)

    part = _E // 16
    off = sid * part
    h1 = pltpu.async_copy(rows_hbm.at[pl.ds(off, part)],
                          shr.at[pl.ds(off, part)], sem_r0)
    h2 = pltpu.async_copy(cols_hbm.at[pl.ds(off, part)],
                          shc.at[pl.ds(off, part)], sem_c0)
    h1.wait()
    h2.wait()
    plsc.subcore_barrier()

    def start_seg(s):
        b = s % 2
        return (pltpu.async_copy(shr.at[pl.ds(s * _SEG, _SEG)],
                                 rbufs[b], sems_r[b]),
                pltpu.async_copy(shc.at[pl.ds(s * _SEG, _SEG)],
                                 cbufs[b], sems_c[b]))

    pend = {0: start_seg(0)}
    out_h = None
    for p in range(2):
        reg = wid + 32 * p
        lo = reg * _RROWS
        if out_h is not None:
            out_h.wait()

        for zr in range(_RROWS):
            def zinit(i, _):
                vbuf[zr, pl.ds(pl.multiple_of(i * 16, 16), 16)] = zeros16
                return 0

            lax.fori_loop(0, _N // 16, zinit, 0, unroll=8)
        for s in range(nseg):
            b = s % 2
            nxt = s + 1 if s + 1 < nseg else (0 if p == 0 else None)
            if nxt is not None and nxt not in pend:
                pend[nxt] = start_seg(nxt)
            hr, hc = pend.pop(s)
            hr.wait()
            hc.wait()
            rbuf, cbuf = rbufs[b], cbufs[b]

            def scan(i, _):
                off = pl.ds(pl.multiple_of(i * 16, 16), 16)
                r = rbuf[off]
                c = cbuf[off]
                sel = (r >= lo) & (r < lo + _RROWS)
                rl = jnp.where(sel, r - lo, 0)
                plsc.store_scatter(vbuf, [rl, c], ones16, mask=sel)
                return 0

            lax.fori_loop(0, _SEG // 16, scan, 0, unroll=8)
        out_h = pltpu.async_copy(vbuf, out_hbm.at[pl.ds(lo, _RROWS), :],
                                 sem_o)
    out_h.wait()


def _edge_mask_sc(edge_index):
    n = _N
    body = functools.partial(
        pl.kernel,
        out_type=jax.ShapeDtypeStruct((n, n), jnp.float32),
        mesh=plsc.VectorSubcoreMesh(core_axis_name="c", subcore_axis_name="s"),
        compiler_params=pltpu.CompilerParams(needs_layout_passes=False),
        scratch_types=[
            pltpu.VMEM((_RROWS, n), jnp.float32),
            pltpu.VMEM((_SEG,), jnp.int32),
            pltpu.VMEM((_SEG,), jnp.int32),
            pltpu.VMEM((_SEG,), jnp.int32),
            pltpu.VMEM((_SEG,), jnp.int32),
            pltpu.VMEM_SHARED((_E,), jnp.int32),
            pltpu.VMEM_SHARED((_E,), jnp.int32),
            pltpu.SemaphoreType.DMA,
            pltpu.SemaphoreType.DMA,
            pltpu.SemaphoreType.DMA,
            pltpu.SemaphoreType.DMA,
            pltpu.SemaphoreType.DMA,
        ],
    )(_edge_scatter_body)
    return body(edge_index[0], edge_index[1])


def kernel(x, Wq, bq, Wk, bk, Wv, bv, Wo, bo, Wg1, bg1, Wg2, bg2, edge_index):
    b, n, d = x.shape
    f32 = jnp.float32
    bf16 = jnp.bfloat16

    q, k, v, col_mask = pl.pallas_call(
        _proj_body,
        grid=(b,),
        in_specs=[
            pl.BlockSpec((1, n, d), lambda i: (i, 0, 0)),
            pl.BlockSpec((d, d), lambda i: (0, 0)),
            pl.BlockSpec((1, d), lambda i: (0, 0)),
            pl.BlockSpec((d, d), lambda i: (0, 0)),
            pl.BlockSpec((1, d), lambda i: (0, 0)),
            pl.BlockSpec((d, d), lambda i: (0, 0)),
            pl.BlockSpec((1, d), lambda i: (0, 0)),
            pl.BlockSpec((d // 2, d), lambda i: (0, 0)),
            pl.BlockSpec((1, d // 2), lambda i: (0, 0)),
            pl.BlockSpec((1, d // 2), lambda i: (0, 0)),
            pl.BlockSpec((1, n), lambda i: (0, 0)),
        ],
        out_specs=[
            pl.BlockSpec((1, n, d), lambda i: (i, 0, 0)),
            pl.BlockSpec((1, n, d), lambda i: (i, 0, 0)),
            pl.BlockSpec((1, n, d), lambda i: (i, 0, 0)),
            pl.BlockSpec((1, 1, n), lambda i: (i, 0, 0)),
        ],
        out_shape=[
            jax.ShapeDtypeStruct((b, n, d), bf16),
            jax.ShapeDtypeStruct((b, n, d), bf16),
            jax.ShapeDtypeStruct((b, n, d), bf16),
            jax.ShapeDtypeStruct((b, 1, n), f32),
        ],
    )(x, Wq, bq.reshape(1, d), Wk, bk.reshape(1, d), Wv, bv.reshape(1, d),
      Wg1, bg1.reshape(1, d // 2), Wg2,
      jnp.broadcast_to(bg2.reshape(1, 1), (1, n)))

    edge_mask = _edge_mask_sc(edge_index)

    nr = n // _TR
    out, sparse_mask = pl.pallas_call(
        _attn_body,
        grid=(b, nr),
        in_specs=[
            pl.BlockSpec((1, _TR, d), lambda i, r: (i, r, 0)),
            pl.BlockSpec((1, n, d), lambda i, r: (i, 0, 0)),
            pl.BlockSpec((1, n, d), lambda i, r: (i, 0, 0)),
            pl.BlockSpec((_TR, n), lambda i, r: (r, 0)),
            pl.BlockSpec((1, 1, n), lambda i, r: (i, 0, 0)),
            pl.BlockSpec((d, d), lambda i, r: (0, 0)),
            pl.BlockSpec((1, d), lambda i, r: (0, 0)),
        ],
        out_specs=[
            pl.BlockSpec((1, _TR, d), lambda i, r: (i, r, 0)),
            pl.BlockSpec((1, _H, _TR, n), lambda i, r: (i, 0, r, 0)),
        ],
        out_shape=[
            jax.ShapeDtypeStruct((b, n, d), f32),
            jax.ShapeDtypeStruct((b, _H, n, n), f32),
        ],
        compiler_params=pltpu.CompilerParams(
            vmem_limit_bytes=100 * 1024 * 1024),
    )(q, k, v, edge_mask, col_mask, Wo, bo.reshape(1, d))

    return out, sparse_mask
